# baseline (speedup 1.0000x reference)
import functools

import jax
import jax.numpy as jnp
from jax import lax
from jax.experimental import pallas as pl
from jax.experimental.pallas import tpu as pltpu
from jax.experimental.pallas import tpu_sc as plsc

NC = 2
NS = 16
LANES = 16
CHUNK = 64
NBUF = 3
PACK = 128 // CHUNK


def _mesh():
    return plsc.VectorSubcoreMesh(
        core_axis_name="c", subcore_axis_name="s",
        num_cores=NC, num_subcores=NS)


def _sc_agg(N, D, R, CH):
    RPC = R // NC
    SLOP = (-N) % (NS * 8)
    ZR = (N + SLOP) // NS
    RW = (N // NS) // 8 * 8
    TAIL = N - NS * RW

    scratch = dict(
        s_shared=pltpu.VMEM_SHARED((N + SLOP, D), jnp.float32),
        src_v=pltpu.VMEM((CH, CHUNK), jnp.int32),
        dst_v=pltpu.VMEM((CH, CHUNK), jnp.int32),
        rows_v=pltpu.VMEM((NBUF, CHUNK, D), jnp.float32),
        sem=pltpu.SemaphoreType.DMA((NBUF,)),
    )

    def body(h_hbm, src_hbm, dst_hbm, z128_hbm, s_out, **scr):
        s_shared, src_v, dst_v, rows_v, sem = (
            scr["s_shared"], scr["src_v"], scr["dst_v"], scr["rows_v"],
            scr["sem"])
        c = lax.axis_index("c")
        s = lax.axis_index("s")

        def start_gather(j, slot):
            pltpu.async_copy(
                h_hbm.at[src_v.at[j, pl.ds(0, CHUNK)]],
                rows_v.at[slot, pl.ds(0, CHUNK)], sem.at[slot])

        def wait_gather(j, slot):
            pltpu.make_async_copy(
                h_hbm.at[src_v.at[j, pl.ds(0, CHUNK)]],
                rows_v.at[slot, pl.ds(0, CHUNK)], sem.at[slot]).wait()

        for rr in range(RPC):
            r = c * RPC + rr
            pltpu.sync_copy(z128_hbm.at[pl.ds(0, CHUNK)], rows_v.at[0])
            zbase = s * ZR
            for t in range(ZR // CHUNK):
                pltpu.sync_copy(rows_v.at[0],
                                s_shared.at[pl.ds(zbase + t * CHUNK, CHUNK)])
            rem = ZR % CHUNK
            if rem:
                pltpu.sync_copy(
                    rows_v.at[0, pl.ds(0, rem)],
                    s_shared.at[pl.ds(zbase + (ZR // CHUNK) * CHUNK, rem)])
            pltpu.sync_copy(src_hbm.at[r, s], src_v)
            pltpu.sync_copy(dst_hbm.at[r, s], dst_v)
            plsc.subcore_barrier()

            for p in range(NBUF - 1):
                start_gather(p, p)

            def chunk_body(j, carry):
                slot = lax.rem(j, NBUF)

                @pl.when(j + NBUF - 1 < CH)
                def _():
                    start_gather(j + NBUF - 1, lax.rem(j + NBUF - 1, NBUF))

                wait_gather(j, slot)
                pltpu.sync_copy(rows_v.at[slot], s_shared.at[dst_v.at[j]],
                                add=True)
                return carry

            lax.fori_loop(0, CH, chunk_body, 0)
            plsc.subcore_barrier()

            pltpu.sync_copy(s_shared.at[pl.ds(s * RW, RW)],
                            s_out.at[r, pl.ds(s * RW, RW)])
            if TAIL:
                @pl.when(s == NS - 1)
                def _():
                    pltpu.sync_copy(s_shared.at[pl.ds(NS * RW, TAIL)],
                                    s_out.at[r, pl.ds(NS * RW, TAIL)])
            plsc.subcore_barrier()

    return pl.kernel(
        body, out_type=jax.ShapeDtypeStruct((R, N, D), jnp.float32),
        mesh=_mesh(), scratch_types=scratch)


def _sc_counts(N, D, R, CH):
    RPC = R // NC
    SLOP = (-N) % (NS * 8)
    ZR = (N + SLOP) // NS
    RW = (N // NS) // 8 * 8
    TAIL = N - NS * RW

    CHP = CH // PACK

    scratch = dict(
        c_shared=pltpu.VMEM_SHARED((N + SLOP, D), jnp.float32),
        dst_v=pltpu.VMEM((CHP, PACK * CHUNK), jnp.int32),
        ones_v=pltpu.VMEM((PACK * CHUNK, D), jnp.float32),
    )

    def body(dst_hbm, z128_hbm, o128_hbm, c_out, **scr):
        c_shared, dst_v, ones_v = (
            scr["c_shared"], scr["dst_v"], scr["ones_v"])
        c = lax.axis_index("c")
        s = lax.axis_index("s")
        CC = PACK * CHUNK

        for rr in range(RPC):
            r = c * RPC + rr
            pltpu.sync_copy(z128_hbm, ones_v)
            zbase = s * ZR
            for t in range(ZR // CC):
                pltpu.sync_copy(ones_v,
                                c_shared.at[pl.ds(zbase + t * CC, CC)])
            rem = ZR % CC
            if rem:
                pltpu.sync_copy(
                    ones_v.at[pl.ds(0, rem)],
                    c_shared.at[pl.ds(zbase + (ZR // CC) * CC, rem)])
            pltpu.sync_copy(dst_hbm.at[r, s], dst_v)
            pltpu.sync_copy(o128_hbm, ones_v)
            plsc.subcore_barrier()

            def chunk_body(j, carry):
                pltpu.sync_copy(ones_v, c_shared.at[dst_v.at[j]], add=True)
                return carry

            lax.fori_loop(0, CHP, chunk_body, 0)
            plsc.subcore_barrier()

            pltpu.sync_copy(c_shared.at[pl.ds(s * RW, RW)],
                            c_out.at[r, pl.ds(s * RW, RW)])
            if TAIL:
                @pl.when(s == NS - 1)
                def _():
                    pltpu.sync_copy(c_shared.at[pl.ds(NS * RW, TAIL)],
                                    c_out.at[r, pl.ds(NS * RW, TAIL)])
            plsc.subcore_barrier()

    return pl.kernel(
        body, out_type=jax.ShapeDtypeStruct((R, N, D), jnp.float32),
        mesh=_mesh(), scratch_types=scratch)


def _dense_body(R, act, h_ref, s_ref, c_ref, wroot_ref, wrel_ref, b_ref,
                sc_ref, beta_ref, out_ref):
    h = h_ref[...]
    acc = jax.lax.dot(h, wroot_ref[...],
                      preferred_element_type=jnp.float32) + b_ref[...]
    for r in range(R):
        inv = 1.0 / jnp.maximum(c_ref[r, :, 0:1], 1.0)
        acc = acc + jax.lax.dot(s_ref[r] * inv, wrel_ref[r],
                                preferred_element_type=jnp.float32)
    if act:
        acc = acc * sc_ref[...] + beta_ref[...]
        acc = jnp.where(acc > 0, acc, jnp.exp(jnp.minimum(acc, 0.0)) - 1.0)
    out_ref[...] = acc


def _dense(N, D_IN, D_OUT, R, act, blk):
    grid = N // blk
    return pl.pallas_call(
        functools.partial(_dense_body, R, act),
        grid=(grid,),
        in_specs=[
            pl.BlockSpec((blk, D_IN), lambda i: (i, 0)),
            pl.BlockSpec((R, blk, D_IN), lambda i: (0, i, 0)),
            pl.BlockSpec((R, blk, D_IN), lambda i: (0, i, 0)),
            pl.BlockSpec((D_IN, D_OUT), lambda i: (0, 0)),
            pl.BlockSpec((R, D_IN, D_OUT), lambda i: (0, 0, 0)),
            pl.BlockSpec((1, D_OUT), lambda i: (0, 0)),
            pl.BlockSpec((1, D_OUT), lambda i: (0, 0)),
            pl.BlockSpec((1, D_OUT), lambda i: (0, 0)),
        ],
        out_specs=pl.BlockSpec((blk, D_OUT), lambda i: (i, 0)),
        out_shape=jax.ShapeDtypeStruct((N, D_OUT), jnp.float32),
    )


def kernel(rel_edges, emb, W_rel0, W_root0, b0, gamma, beta, W_rel1,
           W_root1, b1):
    R, _, E = rel_edges.shape
    N, D = emb.shape
    D_H = W_rel0.shape[2]
    D_OUT = W_rel1.shape[2]

    per_w = -(-E // (NS * PACK * CHUNK)) * PACK * CHUNK
    CH = per_w // CHUNK
    EP = NS * per_w
    src = rel_edges[:, 0, :].astype(jnp.int32)
    dst = rel_edges[:, 1, :].astype(jnp.int32)
    SLOP = (-N) % (NS * 8)
    pad_dst = N + jnp.arange(EP - E, dtype=jnp.int32) % max(SLOP, 1)
    src = jnp.pad(src, ((0, 0), (0, EP - E))).reshape(R, NS, CH, CHUNK)
    dstp = jnp.concatenate(
        [dst, jnp.broadcast_to(pad_dst, (R, EP - E))], axis=1)
    dst = dstp.reshape(R, NS, CH, CHUNK)
    dstw = dstp.reshape(R, NS, CH // PACK, PACK * CHUNK)

    z128 = jnp.zeros((PACK * CHUNK, D), jnp.float32)
    o128 = jnp.ones((PACK * CHUNK, D), jnp.float32)

    bn_scale = (gamma / jnp.sqrt(1.0 + 1e-5)).reshape(1, D_H)
    bn_shift = beta.reshape(1, D_H)
    b0_2 = b0.reshape(1, D_H)
    b1_2 = b1.reshape(1, D_OUT)

    cnt = _sc_counts(N, D, R, CH)(dstw, z128, o128)
    S0 = _sc_agg(N, D, R, CH)(emb, src, dst, z128)
    h1 = _dense(N, D, D_H, R, True, 1000)(
        emb, S0, cnt, W_root0, W_rel0, b0_2, bn_scale, bn_shift)
    S1 = _sc_agg(N, D_H, R, CH)(h1, src, dst, z128)
    out = _dense(N, D_H, D_OUT, R, False, 1000)(
        h1, S1, cnt, W_root1, W_rel1, b1_2, bn_scale, bn_shift)
    return out


# speedup vs baseline: 1.4738x; 1.4738x over previous
import functools

import jax
import jax.numpy as jnp
from jax import lax
from jax.experimental import pallas as pl
from jax.experimental.pallas import tpu as pltpu
from jax.experimental.pallas import tpu_sc as plsc

NC = 2
NS = 16
LANES = 16
CHUNK = 64
NBUF = 3
SPLIT = 1


def _mesh():
    return plsc.VectorSubcoreMesh(
        core_axis_name="c", subcore_axis_name="s",
        num_cores=NC, num_subcores=NS)


def _sc_agg(N, D, R, CH):
    RPC = R // NC
    SLOP = (-N) % (NS * 8)
    ZR = (N + SLOP) // NS
    RW = (N // NS) // 8 * 8
    TAIL = N - NS * RW

    scratch = dict(
        s_shared=pltpu.VMEM_SHARED((N + SLOP, D), jnp.float32),
        src_v=pltpu.VMEM((CH, CHUNK), jnp.int32),
        dst_v=pltpu.VMEM((CH, CHUNK), jnp.int32),
        rows_v=pltpu.VMEM((NBUF, CHUNK, D), jnp.float32),
        sem=pltpu.SemaphoreType.DMA((NBUF, SPLIT)),
    )

    def body(h_hbm, src_hbm, dst_hbm, z128_hbm, s_out, **scr):
        s_shared, src_v, dst_v, rows_v, sem = (
            scr["s_shared"], scr["src_v"], scr["dst_v"], scr["rows_v"],
            scr["sem"])
        c = lax.axis_index("c")
        s = lax.axis_index("s")

        SUB = CHUNK // SPLIT

        def start_gather(j, slot):
            for k in range(SPLIT):
                pltpu.async_copy(
                    h_hbm.at[src_v.at[j, pl.ds(k * SUB, SUB)]],
                    rows_v.at[slot, pl.ds(k * SUB, SUB)], sem.at[slot, k])

        def wait_gather(j, slot):
            for k in range(SPLIT):
                pltpu.make_async_copy(
                    h_hbm.at[src_v.at[j, pl.ds(k * SUB, SUB)]],
                    rows_v.at[slot, pl.ds(k * SUB, SUB)],
                    sem.at[slot, k]).wait()

        for rr in range(RPC):
            r = c * RPC + rr
            pltpu.sync_copy(z128_hbm, rows_v.at[0])
            zbase = s * ZR
            for t in range(ZR // CHUNK):
                pltpu.sync_copy(rows_v.at[0],
                                s_shared.at[pl.ds(zbase + t * CHUNK, CHUNK)])
            rem = ZR % CHUNK
            if rem:
                pltpu.sync_copy(
                    rows_v.at[0, pl.ds(0, rem)],
                    s_shared.at[pl.ds(zbase + (ZR // CHUNK) * CHUNK, rem)])
            pltpu.sync_copy(src_hbm.at[r, s], src_v)
            pltpu.sync_copy(dst_hbm.at[r, s], dst_v)
            plsc.subcore_barrier()

            for p in range(NBUF - 1):
                start_gather(p, p)

            def chunk_body(j, carry):
                slot = lax.rem(j, NBUF)

                @pl.when(j + NBUF - 1 < CH)
                def _():
                    start_gather(j + NBUF - 1, lax.rem(j + NBUF - 1, NBUF))

                wait_gather(j, slot)
                pltpu.sync_copy(rows_v.at[slot], s_shared.at[dst_v.at[j]],
                                add=True)
                return carry

            lax.fori_loop(0, CH, chunk_body, 0)
            plsc.subcore_barrier()

            pltpu.sync_copy(s_shared.at[pl.ds(s * RW, RW)],
                            s_out.at[r, pl.ds(s * RW, RW)])
            if TAIL:
                @pl.when(s == NS - 1)
                def _():
                    pltpu.sync_copy(s_shared.at[pl.ds(NS * RW, TAIL)],
                                    s_out.at[r, pl.ds(NS * RW, TAIL)])
            plsc.subcore_barrier()

    return pl.kernel(
        body, out_type=jax.ShapeDtypeStruct((R, N, D), jnp.float32),
        mesh=_mesh(), scratch_types=scratch)


def _sc_counts(N, D, R, CH):
    RPC = R // NC
    SLOP = (-N) % (NS * 8)
    ZR = (N + SLOP) // NS
    RW = (N // NS) // 8 * 8
    TAIL = N - NS * RW

    scratch = dict(
        c_shared=pltpu.VMEM_SHARED((N + SLOP, D), jnp.float32),
        dst_v=pltpu.VMEM((CH, CHUNK), jnp.int32),
        ones_v=pltpu.VMEM((CHUNK, D), jnp.float32),
    )

    def body(dst_hbm, z128_hbm, o128_hbm, c_out, **scr):
        c_shared, dst_v, ones_v = (
            scr["c_shared"], scr["dst_v"], scr["ones_v"])
        c = lax.axis_index("c")
        s = lax.axis_index("s")

        for rr in range(RPC):
            r = c * RPC + rr
            pltpu.sync_copy(z128_hbm, ones_v)
            zbase = s * ZR
            for t in range(ZR // CHUNK):
                pltpu.sync_copy(ones_v,
                                c_shared.at[pl.ds(zbase + t * CHUNK, CHUNK)])
            rem = ZR % CHUNK
            if rem:
                pltpu.sync_copy(
                    ones_v.at[pl.ds(0, rem)],
                    c_shared.at[pl.ds(zbase + (ZR // CHUNK) * CHUNK, rem)])
            pltpu.sync_copy(dst_hbm.at[r, s], dst_v)
            pltpu.sync_copy(o128_hbm, ones_v)
            plsc.subcore_barrier()

            def chunk_body(j, carry):
                pltpu.sync_copy(ones_v, c_shared.at[dst_v.at[j]], add=True)
                return carry

            lax.fori_loop(0, CH, chunk_body, 0)
            plsc.subcore_barrier()

            pltpu.sync_copy(c_shared.at[pl.ds(s * RW, RW)],
                            c_out.at[r, pl.ds(s * RW, RW)])
            if TAIL:
                @pl.when(s == NS - 1)
                def _():
                    pltpu.sync_copy(c_shared.at[pl.ds(NS * RW, TAIL)],
                                    c_out.at[r, pl.ds(NS * RW, TAIL)])
            plsc.subcore_barrier()

    return pl.kernel(
        body, out_type=jax.ShapeDtypeStruct((R, N, D), jnp.float32),
        mesh=_mesh(), scratch_types=scratch)


def _dense_body(R, act, h_ref, s_ref, c_ref, wroot_ref, wrel_ref, b_ref,
                sc_ref, beta_ref, out_ref):
    h = h_ref[...]
    acc = jax.lax.dot(h, wroot_ref[...],
                      preferred_element_type=jnp.float32) + b_ref[...]
    for r in range(R):
        inv = 1.0 / jnp.maximum(c_ref[r, :, 0:1], 1.0)
        acc = acc + jax.lax.dot(s_ref[r] * inv, wrel_ref[r],
                                preferred_element_type=jnp.float32)
    if act:
        acc = acc * sc_ref[...] + beta_ref[...]
        acc = jnp.where(acc > 0, acc, jnp.exp(jnp.minimum(acc, 0.0)) - 1.0)
    out_ref[...] = acc


def _dense(N, D_IN, D_OUT, R, act, blk):
    grid = N // blk
    return pl.pallas_call(
        functools.partial(_dense_body, R, act),
        grid=(grid,),
        in_specs=[
            pl.BlockSpec((blk, D_IN), lambda i: (i, 0)),
            pl.BlockSpec((R, blk, D_IN), lambda i: (0, i, 0)),
            pl.BlockSpec((R, blk, D_IN), lambda i: (0, i, 0)),
            pl.BlockSpec((D_IN, D_OUT), lambda i: (0, 0)),
            pl.BlockSpec((R, D_IN, D_OUT), lambda i: (0, 0, 0)),
            pl.BlockSpec((1, D_OUT), lambda i: (0, 0)),
            pl.BlockSpec((1, D_OUT), lambda i: (0, 0)),
            pl.BlockSpec((1, D_OUT), lambda i: (0, 0)),
        ],
        out_specs=pl.BlockSpec((blk, D_OUT), lambda i: (i, 0)),
        out_shape=jax.ShapeDtypeStruct((N, D_OUT), jnp.float32),
    )


def kernel(rel_edges, emb, W_rel0, W_root0, b0, gamma, beta, W_rel1,
           W_root1, b1):
    R, _, E = rel_edges.shape
    N, D = emb.shape
    D_H = W_rel0.shape[2]
    D_OUT = W_rel1.shape[2]

    per_w = -(-E // (NS * CHUNK)) * CHUNK
    CH = per_w // CHUNK
    EP = NS * per_w
    src = rel_edges[:, 0, :].astype(jnp.int32)
    dst = rel_edges[:, 1, :].astype(jnp.int32)
    src = jnp.pad(src, ((0, 0), (0, EP - E))).reshape(R, NS, CH, CHUNK)
    dst = jnp.pad(dst, ((0, 0), (0, EP - E)),
                  constant_values=N).reshape(R, NS, CH, CHUNK)

    z128 = jnp.zeros((CHUNK, D), jnp.float32)
    o128 = jnp.ones((CHUNK, D), jnp.float32)

    bn_scale = (gamma / jnp.sqrt(1.0 + 1e-5)).reshape(1, D_H)
    bn_shift = beta.reshape(1, D_H)
    b0_2 = b0.reshape(1, D_H)
    b1_2 = b1.reshape(1, D_OUT)

    cnt = _sc_counts(N, D, R, CH)(dst, z128, o128)
    S0 = _sc_agg(N, D, R, CH)(emb, src, dst, z128)
    h1 = _dense(N, D, D_H, R, True, 1000)(
        emb, S0, cnt, W_root0, W_rel0, b0_2, bn_scale, bn_shift)
    S1 = _sc_agg(N, D_H, R, CH)(h1, src, dst, z128)
    out = _dense(N, D_H, D_OUT, R, False, 1000)(
        h1, S1, cnt, W_root1, W_rel1, b1_2, bn_scale, bn_shift)
    return out
